# Initial kernel scaffold; baseline (speedup 1.0000x reference)
import jax
import jax.numpy as jnp
from jax.experimental import pallas as pl


def kernel(points, point_features, centroids, centroid_features, W1, b1, gamma, beta, W2, b2):
    raise NotImplementedError("write your pallas kernel here")


# speedup vs baseline: 17.1946x; 17.1946x over previous
import functools

import jax
import jax.numpy as jnp
from jax import lax
from jax.experimental import pallas as pl
from jax.experimental.pallas import tpu as pltpu
from jax.experimental.pallas import tpu_sc as plsc

B = 8
N = 4096
M = 1024
CIN = 128
SKIP = 128
COUT = 128
K = 3
P = B * N

TN = 512
NT = N // TN

NW = 32
CHUNK = P // NW
G = 128
NG = CHUNK // G


def _knn_body(pts_ref, cent_ref, gidx_ref, wn_ref):
    b = pl.program_id(0)
    pts = pts_ref[0]
    cent = cent_ref[0]
    acc = jnp.zeros((TN, M), jnp.float32)
    for d in range(3):
        diff = pts[:, d:d + 1] - cent[d:d + 1, :]
        acc = acc + diff * diff
    col = lax.broadcasted_iota(jnp.int32, (TN, M), 1)
    work = acc
    idxs, vals = [], []
    for _ in range(K):
        mval = jnp.min(work, axis=1, keepdims=True)
        cand = jnp.where(work == mval, col, M)
        ik = jnp.min(cand, axis=1, keepdims=True)
        idxs.append(ik)
        vals.append(mval)
        work = jnp.where(col == ik, jnp.float32(jnp.inf), work)
    ws = [1.0 / jnp.maximum(v, 1e-16) for v in vals]
    wt = ws[0] + ws[1] + ws[2]
    zi = jnp.zeros((TN, 8 - K), jnp.int32)
    zf = jnp.zeros((TN, 8 - K), jnp.float32)
    gidx_ref[0] = jnp.concatenate([i + b * M for i in idxs] + [zi], axis=1)
    wn_ref[0] = jnp.concatenate([w / wt for w in ws] + [zf], axis=1)


def _knn(ptsT8, cent8):
    return pl.pallas_call(
        _knn_body,
        grid=(B, NT),
        in_specs=[
            pl.BlockSpec((1, TN, 8), lambda b, t: (b, t, 0)),
            pl.BlockSpec((1, 8, M), lambda b, t: (b, 0, 0)),
        ],
        out_specs=[
            pl.BlockSpec((1, TN, 8), lambda b, t: (b, t, 0)),
            pl.BlockSpec((1, TN, 8), lambda b, t: (b, t, 0)),
        ],
        out_shape=[
            jax.ShapeDtypeStruct((B, N, 8), jnp.int32),
            jax.ShapeDtypeStruct((B, N, 8), jnp.float32),
        ],
    )(ptsT8, cent8)


def _sc_gather_body(cf_hbm, g0_hbm, g1_hbm, g2_hbm,
                    o0_hbm, o1_hbm, o2_hbm,
                    i0, i1, i2, buf0, buf1, buf2, s0, s1, s2):
    wid = lax.axis_index("s") * 2 + lax.axis_index("c")
    pltpu.sync_copy(g0_hbm.at[wid], i0)
    pltpu.sync_copy(g1_hbm.at[wid], i1)
    pltpu.sync_copy(g2_hbm.at[wid], i2)
    base = wid * CHUNK
    for g in range(NG):
        c0 = pltpu.async_copy(cf_hbm.at[i0.at[g]], buf0, s0)
        c1 = pltpu.async_copy(cf_hbm.at[i1.at[g]], buf1, s1)
        c2 = pltpu.async_copy(cf_hbm.at[i2.at[g]], buf2, s2)
        c0.wait()
        c1.wait()
        c2.wait()
        row = base + g * G
        pltpu.sync_copy(buf0, o0_hbm.at[pl.ds(row, G)])
        pltpu.sync_copy(buf1, o1_hbm.at[pl.ds(row, G)])
        pltpu.sync_copy(buf2, o2_hbm.at[pl.ds(row, G)])


def _sc_gather(cf_flat, g0, g1, g2):
    f32 = jnp.float32
    call = pl.kernel(
        _sc_gather_body,
        out_type=[
            jax.ShapeDtypeStruct((P, CIN), f32),
            jax.ShapeDtypeStruct((P, CIN), f32),
            jax.ShapeDtypeStruct((P, CIN), f32),
        ],
        mesh=plsc.VectorSubcoreMesh(core_axis_name="c", subcore_axis_name="s"),
        scratch_types=[
            pltpu.VMEM((NG, G), jnp.int32),
            pltpu.VMEM((NG, G), jnp.int32),
            pltpu.VMEM((NG, G), jnp.int32),
            pltpu.VMEM((G, CIN), f32),
            pltpu.VMEM((G, CIN), f32),
            pltpu.VMEM((G, CIN), f32),
            pltpu.SemaphoreType.DMA,
            pltpu.SemaphoreType.DMA,
            pltpu.SemaphoreType.DMA,
        ],
    )
    return call(cf_flat, g0, g1, g2)


def _mlp1_body(pf_ref, g0_ref, g1_ref, g2_ref, wn_ref, W1a_ref, W1b_ref,
               b1_ref, h_ref, st_ref):
    wn = wn_ref[0]
    interp = (wn[:, 0:1] * g0_ref[...] +
              wn[:, 1:2] * g1_ref[...] +
              wn[:, 2:3] * g2_ref[...])
    hT = lax.dot_general(pf_ref[0], W1a_ref[...], (((0,), (1,)), ((), ())),
                         preferred_element_type=jnp.float32)
    hT = hT + lax.dot_general(interp, W1b_ref[...], (((1,), (1,)), ((), ())),
                              preferred_element_type=jnp.float32)
    hT = hT + b1_ref[...]
    h_ref[...] = hT
    s = jnp.sum(hT, axis=0, keepdims=True)
    q = jnp.sum(hT * hT, axis=0, keepdims=True)
    ri = lax.broadcasted_iota(jnp.int32, (8, COUT), 0)
    upd = (jnp.where(ri == 0, jnp.broadcast_to(s, (8, COUT)), 0.0) +
           jnp.where(ri == 1, jnp.broadcast_to(q, (8, COUT)), 0.0))
    first = jnp.logical_and(pl.program_id(0) == 0, pl.program_id(1) == 0)

    @pl.when(first)
    def _():
        st_ref[...] = upd

    @pl.when(jnp.logical_not(first))
    def _():
        st_ref[...] = st_ref[...] + upd


def _mlp1(pf, o0, o1, o2, wn, W1a, W1b, b1row):
    return pl.pallas_call(
        _mlp1_body,
        grid=(B, NT),
        in_specs=[
            pl.BlockSpec((1, SKIP, TN), lambda b, t: (b, 0, t)),
            pl.BlockSpec((TN, CIN), lambda b, t: (b * NT + t, 0)),
            pl.BlockSpec((TN, CIN), lambda b, t: (b * NT + t, 0)),
            pl.BlockSpec((TN, CIN), lambda b, t: (b * NT + t, 0)),
            pl.BlockSpec((1, TN, 8), lambda b, t: (b, t, 0)),
            pl.BlockSpec((COUT, SKIP), lambda b, t: (0, 0)),
            pl.BlockSpec((COUT, CIN), lambda b, t: (0, 0)),
            pl.BlockSpec((1, COUT), lambda b, t: (0, 0)),
        ],
        out_specs=[
            pl.BlockSpec((TN, COUT), lambda b, t: (b * NT + t, 0)),
            pl.BlockSpec((8, COUT), lambda b, t: (0, 0)),
        ],
        out_shape=[
            jax.ShapeDtypeStruct((P, COUT), jnp.float32),
            jax.ShapeDtypeStruct((8, COUT), jnp.float32),
        ],
    )(pf, o0, o1, o2, wn, W1a, W1b, b1row)


def _mlp2_body(h_ref, st_ref, gam_ref, bet_ref, W2_ref, b2_ref, out_ref):
    inv_p = jnp.float32(1.0 / P)
    mean = st_ref[0:1, :] * inv_p
    ex2 = st_ref[1:2, :] * inv_p
    var = ex2 - mean * mean
    rstd = lax.rsqrt(var + 1e-5)
    scale = gam_ref[...] * rstd
    shift = bet_ref[...] - mean * scale
    hr = jnp.maximum(h_ref[...] * scale + shift, 0.0)
    o = lax.dot_general(W2_ref[...], hr, (((1,), (1,)), ((), ())),
                        preferred_element_type=jnp.float32)
    out_ref[0] = o + b2_ref[...]


def _mlp2(h, st, gamma_row, beta_row, W2, b2col):
    return pl.pallas_call(
        _mlp2_body,
        grid=(B, NT),
        in_specs=[
            pl.BlockSpec((TN, COUT), lambda b, t: (b * NT + t, 0)),
            pl.BlockSpec((8, COUT), lambda b, t: (0, 0)),
            pl.BlockSpec((1, COUT), lambda b, t: (0, 0)),
            pl.BlockSpec((1, COUT), lambda b, t: (0, 0)),
            pl.BlockSpec((COUT, COUT), lambda b, t: (0, 0)),
            pl.BlockSpec((COUT, 1), lambda b, t: (0, 0)),
        ],
        out_specs=pl.BlockSpec((1, COUT, TN), lambda b, t: (b, 0, t)),
        out_shape=jax.ShapeDtypeStruct((B, COUT, N), jnp.float32),
    )(h, st, gamma_row, beta_row, W2, b2col)


def kernel(points, point_features, centroids, centroid_features,
           W1, b1, gamma, beta, W2, b2):
    f32 = jnp.float32
    ptsT8 = jnp.pad(jnp.transpose(points, (0, 2, 1)), ((0, 0), (0, 0), (0, 5)))
    cent8 = jnp.pad(centroids, ((0, 0), (0, 5), (0, 0)))
    gidx, wn = _knn(ptsT8, cent8)

    cf_flat = jnp.transpose(centroid_features, (0, 2, 1)).reshape(B * M, CIN)
    g0 = gidx[:, :, 0].reshape(NW, NG, G)
    g1 = gidx[:, :, 1].reshape(NW, NG, G)
    g2 = gidx[:, :, 2].reshape(NW, NG, G)
    o0, o1, o2 = _sc_gather(cf_flat, g0, g1, g2)

    W1a = W1[:, :SKIP]
    W1b = W1[:, SKIP:]
    h, st = _mlp1(point_features, o0, o1, o2, wn, W1a, W1b,
                  b1.reshape(1, COUT).astype(f32))
    out = _mlp2(h, st, gamma.reshape(1, COUT), beta.reshape(1, COUT),
                W2, b2.reshape(COUT, 1))
    return out
